# Initial kernel scaffold; baseline (speedup 1.0000x reference)
import jax
import jax.numpy as jnp
from jax.experimental import pallas as pl


def kernel(feats, edge_index, etypes, node_graph_ids, W_et, b_et, gru_Wih, gru_Whh, gru_bih, gru_bhh, lstm_Wih0, lstm_Whh0, lstm_bih0, lstm_bhh0, lstm_Wih1, lstm_Whh1, lstm_bih1, lstm_bhh1, lstm_Wih2, lstm_Whh2, lstm_bih2, lstm_bhh2, W_pred, b_pred):
    raise NotImplementedError("write your pallas kernel here")


# speedup vs baseline: 25.6796x; 25.6796x over previous
import functools

import jax
import jax.numpy as jnp
from jax import lax
from jax.experimental import pallas as pl
from jax.experimental.pallas import tpu as pltpu, tpu_sc as plsc

N = 10000
E = 320000
F = 128
K = 6
N_STEPS = 5
N_ITERS = 3
BATCH = 64
HID = F

NC = 2
NS = 16
NW = NC * NS
EPT = E // NW
CH = 125
NCH = EPT // CH
RPT = 632
N_PAD = RPT * NS

_f32 = jnp.float32


def _hall_body(h_ref, w_ref, b_ref, out_ref):
    out_ref[...] = (
        jnp.dot(h_ref[...], w_ref[...], preferred_element_type=_f32)
        + b_ref[...]
    )


def _hall(h, w_cat, b_cat, bn=1000):
    return pl.pallas_call(
        _hall_body,
        grid=(N // bn,),
        in_specs=[
            pl.BlockSpec((bn, F), lambda i: (i, 0)),
            pl.BlockSpec((F, K * F), lambda i: (0, 0)),
            pl.BlockSpec((1, K * F), lambda i: (0, 0)),
        ],
        out_specs=pl.BlockSpec((bn, K * F), lambda i: (i, 0)),
        out_shape=jax.ShapeDtypeStruct((N, K * F), _f32),
    )(h, w_cat, b_cat)


def _edge_body(tbl, gidx2d, didx2d, zeros, out,
               gidx_all, didx_all, rows, acc, sem):
    c = lax.axis_index("c")
    s = lax.axis_index("s")
    wid = s * NC + c

    r0 = pl.multiple_of(s * RPT, 8)
    pltpu.sync_copy(zeros, acc.at[pl.ds(r0, RPT), :])

    i0 = pl.multiple_of(wid * NCH, 8)
    pltpu.sync_copy(gidx2d.at[pl.ds(i0, NCH), :], gidx_all)
    pltpu.sync_copy(didx2d.at[pl.ds(i0, NCH), :], didx_all)
    plsc.subcore_barrier()

    def chunk(j, carry):
        pltpu.async_copy(tbl.at[gidx_all.at[j]], rows, sem).wait()
        pltpu.sync_copy(rows, acc.at[didx_all.at[j]], add=True)
        return carry

    lax.fori_loop(0, NCH, chunk, 0)
    plsc.subcore_barrier()

    pltpu.sync_copy(acc.at[pl.ds(r0, RPT), :], out.at[c, pl.ds(r0, RPT), :])


@functools.cache
def _edge_pass_built():
    return functools.partial(
        pl.kernel,
        out_type=jax.ShapeDtypeStruct((NC, N_PAD, F), _f32),
        mesh=plsc.VectorSubcoreMesh(
            core_axis_name="c", subcore_axis_name="s",
            num_cores=NC, num_subcores=NS,
        ),
        scratch_types=[
            pltpu.VMEM((NCH, CH), jnp.int32),
            pltpu.VMEM((NCH, CH), jnp.int32),
            pltpu.VMEM((CH, F), _f32),
            pltpu.VMEM_SHARED((N_PAD, F), _f32),
            pltpu.SemaphoreType.DMA,
        ],
    )(_edge_body)


def _edge_pass(tbl, gidx2d, didx2d, zeros):
    return _edge_pass_built()(tbl, gidx2d, didx2d, zeros)


def _gru_body(a_ref, h_ref, wi_ref, wh_ref, bi_ref, bh_ref, out_ref):
    a = a_ref[0] + a_ref[1]
    h = h_ref[...]
    gi = jnp.dot(a, wi_ref[...], preferred_element_type=_f32) + bi_ref[...]
    gh = jnp.dot(h, wh_ref[...], preferred_element_type=_f32) + bh_ref[...]
    r = jax.nn.sigmoid(gi[:, :F] + gh[:, :F])
    z = jax.nn.sigmoid(gi[:, F:2 * F] + gh[:, F:2 * F])
    n = jnp.tanh(gi[:, 2 * F:] + r * gh[:, 2 * F:])
    out_ref[...] = (1.0 - z) * n + z * h


def _gru(parts, h, wiT, whT, bi, bh, bn=1000):
    return pl.pallas_call(
        _gru_body,
        grid=(N // bn,),
        in_specs=[
            pl.BlockSpec((NC, bn, F), lambda i: (0, i, 0)),
            pl.BlockSpec((bn, F), lambda i: (i, 0)),
            pl.BlockSpec((F, 3 * F), lambda i: (0, 0)),
            pl.BlockSpec((F, 3 * F), lambda i: (0, 0)),
            pl.BlockSpec((1, 3 * F), lambda i: (0, 0)),
            pl.BlockSpec((1, 3 * F), lambda i: (0, 0)),
        ],
        out_specs=pl.BlockSpec((bn, F), lambda i: (i, 0)),
        out_shape=jax.ShapeDtypeStruct((N, F), _f32),
    )(parts, h, wiT, whT, bi, bh)


def _s2s_body(feat_ref, ids_ref, wi0, wh0, b0, wi1, wh1, b1, wi2, wh2, b2,
              wpredT, bpred, out_ref):
    feat = feat_ref[...]
    ids = ids_ref[...]
    gids = lax.broadcasted_iota(jnp.int32, (N, BATCH), 1)
    pb = ids == gids
    pf = pb.astype(_f32)
    lstm = [(wi0, wh0, b0), (wi1, wh1, b1), (wi2, wh2, b2)]

    hs = [jnp.zeros((BATCH, HID), _f32) for _ in range(3)]
    cs = [jnp.zeros((BATCH, HID), _f32) for _ in range(3)]
    q_star = jnp.zeros((BATCH, 2 * HID), _f32)
    for _ in range(N_ITERS):
        x = q_star
        for l in range(3):
            wi, wh, b = lstm[l]
            gates = (
                jnp.dot(x, wi[...], preferred_element_type=_f32)
                + jnp.dot(hs[l], wh[...], preferred_element_type=_f32)
                + b[...]
            )
            gi = gates[:, :HID]
            gf = gates[:, HID:2 * HID]
            gg = gates[:, 2 * HID:3 * HID]
            go = gates[:, 3 * HID:]
            cs[l] = jax.nn.sigmoid(gf) * cs[l] + jax.nn.sigmoid(gi) * jnp.tanh(gg)
            hs[l] = jax.nn.sigmoid(go) * jnp.tanh(cs[l])
            x = hs[l]
        q = x
        qn = jnp.dot(pf, q, preferred_element_type=_f32)
        e = jnp.sum(feat * qn, axis=1, keepdims=True)
        em = jnp.max(jnp.where(pb, e, -1e30), axis=0, keepdims=True)
        emg = lax.dot_general(pf, em, (((1,), (1,)), ((), ())),
                              preferred_element_type=_f32)
        ee = jnp.exp(e - emg)
        den = lax.dot_general(ee, pf, (((0,), (0,)), ((), ())),
                              preferred_element_type=_f32)
        deng = lax.dot_general(pf, den, (((1,), (1,)), ((), ())),
                               preferred_element_type=_f32)
        alpha = ee / deng
        ro = lax.dot_general(pf, feat * alpha, (((0,), (0,)), ((), ())),
                             preferred_element_type=_f32)
        q_star = jnp.concatenate([q, ro], axis=-1)
    out_ref[...] = (
        jnp.dot(q_star, wpredT[...], preferred_element_type=_f32) + bpred[...]
    )


def _s2s(feat, ids2d, lstm_wT, lstm_b, wpredT, bpred):
    args = [feat, ids2d]
    specs = [
        pl.BlockSpec((N, F), lambda: (0, 0)),
        pl.BlockSpec((N, 1), lambda: (0, 0)),
    ]
    in_sizes = [2 * HID, HID, HID]
    for l in range(3):
        args += [lstm_wT[l][0], lstm_wT[l][1], lstm_b[l]]
        specs += [
            pl.BlockSpec((in_sizes[l], 4 * HID), lambda: (0, 0)),
            pl.BlockSpec((HID, 4 * HID), lambda: (0, 0)),
            pl.BlockSpec((1, 4 * HID), lambda: (0, 0)),
        ]
    args += [wpredT, bpred]
    specs += [
        pl.BlockSpec((2 * HID, 3), lambda: (0, 0)),
        pl.BlockSpec((1, 3), lambda: (0, 0)),
    ]
    return pl.pallas_call(
        _s2s_body,
        grid=(),
        in_specs=specs,
        out_specs=pl.BlockSpec((BATCH, 3), lambda: (0, 0)),
        out_shape=jax.ShapeDtypeStruct((BATCH, 3), _f32),
    )(*args)


def kernel(feats, edge_index, etypes, node_graph_ids, W_et, b_et,
           gru_Wih, gru_Whh, gru_bih, gru_bhh,
           lstm_Wih0, lstm_Whh0, lstm_bih0, lstm_bhh0,
           lstm_Wih1, lstm_Whh1, lstm_bih1, lstm_bhh1,
           lstm_Wih2, lstm_Whh2, lstm_bih2, lstm_bhh2,
           W_pred, b_pred):
    src = edge_index[0].astype(jnp.int32)
    dst = edge_index[1].astype(jnp.int32)
    row_idx = (src * K + etypes.astype(jnp.int32)).reshape(E // CH, CH)
    dst2d = dst.reshape(E // CH, CH)

    w_cat = jnp.transpose(W_et, (2, 0, 1)).reshape(F, K * F)
    b_cat = b_et.reshape(1, K * F)
    wiT = gru_Wih.T
    whT = gru_Whh.T
    bi = gru_bih.reshape(1, 3 * F)
    bh = gru_bhh.reshape(1, 3 * F)
    zeros_nf = jnp.zeros((RPT, F), _f32)

    h = feats
    for _ in range(N_STEPS):
        h_all = _hall(h, w_cat, b_cat).reshape(N * K, F)
        parts = _edge_pass(h_all, row_idx, dst2d, zeros_nf)
        h = _gru(parts, h, wiT, whT, bi, bh)

    lstm_wT = [(lstm_Wih0.T, lstm_Whh0.T), (lstm_Wih1.T, lstm_Whh1.T),
               (lstm_Wih2.T, lstm_Whh2.T)]
    lstm_b = [(lstm_bih0 + lstm_bhh0).reshape(1, 4 * HID),
              (lstm_bih1 + lstm_bhh1).reshape(1, 4 * HID),
              (lstm_bih2 + lstm_bhh2).reshape(1, 4 * HID)]
    ids2d = node_graph_ids.astype(jnp.int32).reshape(N, 1)
    return _s2s(h, ids2d, lstm_wT, lstm_b, W_pred.T, b_pred.reshape(1, 3))


# speedup vs baseline: 29.1291x; 1.1343x over previous
import functools

import jax
import jax.numpy as jnp
from jax import lax
from jax.experimental import pallas as pl
from jax.experimental.pallas import tpu as pltpu, tpu_sc as plsc

N = 10000
E = 320000
F = 128
K = 6
N_STEPS = 5
N_ITERS = 3
BATCH = 64
HID = F

NC = 2
NS = 16
NW = NC * NS
CHB = 128
NCHUNK = E // CHB
NCHT = NCHUNK // NW
REM = NCHUNK - NCHT * NW
NPAIR = NCHT // 2
RPT = 632
N_PAD = RPT * NS

_f32 = jnp.float32


def _hall_body(h_ref, w_ref, b_ref, out_ref):
    out_ref[...] = (
        jnp.dot(h_ref[...], w_ref[0], preferred_element_type=_f32)
        + b_ref[0]
    )


def _hall(h, w_kT, b_et2, bn=2000):
    return pl.pallas_call(
        _hall_body,
        grid=(K, N // bn),
        in_specs=[
            pl.BlockSpec((bn, F), lambda e, i: (i, 0)),
            pl.BlockSpec((1, F, F), lambda e, i: (e, 0, 0)),
            pl.BlockSpec((1, 1, F), lambda e, i: (e, 0, 0)),
        ],
        out_specs=pl.BlockSpec((bn, F), lambda e, i: (e * (N // bn) + i, 0)),
        out_shape=jax.ShapeDtypeStruct((K * N, F), _f32),
    )(h, w_kT, b_et2)


def _edge_body(tbl, gidx2d, didx2d, zeros, out,
               gia, dia, gib, dib, rows_a, rows_b,
               iga, ida, igb, idb, gsa, gsb, ssa, ssb, acc):
    c = lax.axis_index("c")
    s = lax.axis_index("s")
    wid = s * NC + c

    pltpu.async_copy(gidx2d.at[wid], gia, iga)
    pltpu.async_copy(didx2d.at[wid], dia, ida)
    pltpu.async_copy(gidx2d.at[wid + NW], gib, igb)
    pltpu.async_copy(didx2d.at[wid + NW], dib, idb)

    r0 = pl.multiple_of(s * RPT, 8)
    pltpu.sync_copy(zeros, acc.at[pl.ds(r0, RPT), :])
    plsc.subcore_barrier()

    def _drain(dst, sem):
        pltpu.make_async_copy(gidx2d.at[0], dst, sem).wait()

    def pair(j, carry):
        _drain(gia, iga)
        ga = pltpu.async_copy(tbl.at[gia], rows_a, gsa)
        _drain(gib, igb)
        gb = pltpu.async_copy(tbl.at[gib], rows_b, gsb)
        _drain(dia, ida)
        ga.wait()
        sa = pltpu.async_copy(rows_a, acc.at[dia], ssa, add=True)
        _drain(dib, idb)
        gb.wait()
        sb = pltpu.async_copy(rows_b, acc.at[dib], ssb, add=True)
        sa.wait()

        @pl.when(j < NPAIR - 1)
        def _():
            cn = wid + NW * (2 * j + 2)
            pltpu.async_copy(gidx2d.at[cn], gia, iga)
            pltpu.async_copy(didx2d.at[cn], dia, ida)

        sb.wait()

        @pl.when(j < NPAIR - 1)
        def _():
            cn = wid + NW * (2 * j + 3)
            pltpu.async_copy(gidx2d.at[cn], gib, igb)
            pltpu.async_copy(didx2d.at[cn], dib, idb)

        return carry

    lax.fori_loop(0, NPAIR, pair, 0)

    @pl.when(wid < REM)
    def _():
        ce = NCHT * NW + wid
        pltpu.sync_copy(gidx2d.at[ce], gia)
        pltpu.sync_copy(didx2d.at[ce], dia)
        pltpu.async_copy(tbl.at[gia], rows_a, gsa).wait()
        pltpu.sync_copy(rows_a, acc.at[dia], add=True)

    plsc.subcore_barrier()

    pltpu.sync_copy(acc.at[pl.ds(r0, RPT), :], out.at[c, pl.ds(r0, RPT), :])


@functools.cache
def _edge_pass_built():
    return functools.partial(
        pl.kernel,
        out_type=jax.ShapeDtypeStruct((NC, N_PAD, F), _f32),
        mesh=plsc.VectorSubcoreMesh(
            core_axis_name="c", subcore_axis_name="s",
            num_cores=NC, num_subcores=NS,
        ),
        scratch_types=[
            pltpu.VMEM((CHB,), jnp.int32),
            pltpu.VMEM((CHB,), jnp.int32),
            pltpu.VMEM((CHB,), jnp.int32),
            pltpu.VMEM((CHB,), jnp.int32),
            pltpu.VMEM((CHB, F), _f32),
            pltpu.VMEM((CHB, F), _f32),
            pltpu.SemaphoreType.DMA,
            pltpu.SemaphoreType.DMA,
            pltpu.SemaphoreType.DMA,
            pltpu.SemaphoreType.DMA,
            pltpu.SemaphoreType.DMA,
            pltpu.SemaphoreType.DMA,
            pltpu.SemaphoreType.DMA,
            pltpu.SemaphoreType.DMA,
            pltpu.VMEM_SHARED((N_PAD, F), _f32),
        ],
    )(_edge_body)


def _edge_pass(tbl, gidx2d, didx2d, zeros):
    return _edge_pass_built()(tbl, gidx2d, didx2d, zeros)


def _gru_body(a_ref, h_ref, wi_ref, wh_ref, bi_ref, bh_ref, out_ref):
    a = a_ref[0] + a_ref[1]
    h = h_ref[...]
    gi = jnp.dot(a, wi_ref[...], preferred_element_type=_f32) + bi_ref[...]
    gh = jnp.dot(h, wh_ref[...], preferred_element_type=_f32) + bh_ref[...]
    r = jax.nn.sigmoid(gi[:, :F] + gh[:, :F])
    z = jax.nn.sigmoid(gi[:, F:2 * F] + gh[:, F:2 * F])
    n = jnp.tanh(gi[:, 2 * F:] + r * gh[:, 2 * F:])
    out_ref[...] = (1.0 - z) * n + z * h


def _gru(parts, h, wiT, whT, bi, bh, bn=1000):
    return pl.pallas_call(
        _gru_body,
        grid=(N // bn,),
        in_specs=[
            pl.BlockSpec((NC, bn, F), lambda i: (0, i, 0)),
            pl.BlockSpec((bn, F), lambda i: (i, 0)),
            pl.BlockSpec((F, 3 * F), lambda i: (0, 0)),
            pl.BlockSpec((F, 3 * F), lambda i: (0, 0)),
            pl.BlockSpec((1, 3 * F), lambda i: (0, 0)),
            pl.BlockSpec((1, 3 * F), lambda i: (0, 0)),
        ],
        out_specs=pl.BlockSpec((bn, F), lambda i: (i, 0)),
        out_shape=jax.ShapeDtypeStruct((N, F), _f32),
    )(parts, h, wiT, whT, bi, bh)


def _s2s_body(feat_ref, ids_ref, wi0, wh0, b0, wi1, wh1, b1, wi2, wh2, b2,
              wpredT, bpred, out_ref):
    feat = feat_ref[...]
    ids = ids_ref[...]
    gids = lax.broadcasted_iota(jnp.int32, (N, BATCH), 1)
    pb = ids == gids
    pf = pb.astype(_f32)
    lstm = [(wi0, wh0, b0), (wi1, wh1, b1), (wi2, wh2, b2)]

    hs = [jnp.zeros((BATCH, HID), _f32) for _ in range(3)]
    cs = [jnp.zeros((BATCH, HID), _f32) for _ in range(3)]
    q_star = jnp.zeros((BATCH, 2 * HID), _f32)
    for _ in range(N_ITERS):
        x = q_star
        for l in range(3):
            wi, wh, b = lstm[l]
            gates = (
                jnp.dot(x, wi[...], preferred_element_type=_f32)
                + jnp.dot(hs[l], wh[...], preferred_element_type=_f32)
                + b[...]
            )
            gi = gates[:, :HID]
            gf = gates[:, HID:2 * HID]
            gg = gates[:, 2 * HID:3 * HID]
            go = gates[:, 3 * HID:]
            cs[l] = jax.nn.sigmoid(gf) * cs[l] + jax.nn.sigmoid(gi) * jnp.tanh(gg)
            hs[l] = jax.nn.sigmoid(go) * jnp.tanh(cs[l])
            x = hs[l]
        q = x
        qn = jnp.dot(pf, q, preferred_element_type=_f32)
        e = jnp.sum(feat * qn, axis=1, keepdims=True)
        em = jnp.max(jnp.where(pb, e, -1e30), axis=0, keepdims=True)
        emg = lax.dot_general(pf, em, (((1,), (1,)), ((), ())),
                              preferred_element_type=_f32)
        ee = jnp.exp(e - emg)
        den = lax.dot_general(ee, pf, (((0,), (0,)), ((), ())),
                              preferred_element_type=_f32)
        deng = lax.dot_general(pf, den, (((1,), (1,)), ((), ())),
                               preferred_element_type=_f32)
        alpha = ee / deng
        ro = lax.dot_general(pf, feat * alpha, (((0,), (0,)), ((), ())),
                             preferred_element_type=_f32)
        q_star = jnp.concatenate([q, ro], axis=-1)
    out_ref[...] = (
        jnp.dot(q_star, wpredT[...], preferred_element_type=_f32) + bpred[...]
    )


def _s2s(feat, ids2d, lstm_wT, lstm_b, wpredT, bpred):
    args = [feat, ids2d]
    specs = [
        pl.BlockSpec((N, F), lambda: (0, 0)),
        pl.BlockSpec((N, 1), lambda: (0, 0)),
    ]
    in_sizes = [2 * HID, HID, HID]
    for l in range(3):
        args += [lstm_wT[l][0], lstm_wT[l][1], lstm_b[l]]
        specs += [
            pl.BlockSpec((in_sizes[l], 4 * HID), lambda: (0, 0)),
            pl.BlockSpec((HID, 4 * HID), lambda: (0, 0)),
            pl.BlockSpec((1, 4 * HID), lambda: (0, 0)),
        ]
    args += [wpredT, bpred]
    specs += [
        pl.BlockSpec((2 * HID, 3), lambda: (0, 0)),
        pl.BlockSpec((1, 3), lambda: (0, 0)),
    ]
    return pl.pallas_call(
        _s2s_body,
        grid=(),
        in_specs=specs,
        out_specs=pl.BlockSpec((BATCH, 3), lambda: (0, 0)),
        out_shape=jax.ShapeDtypeStruct((BATCH, 3), _f32),
    )(*args)


def kernel(feats, edge_index, etypes, node_graph_ids, W_et, b_et,
           gru_Wih, gru_Whh, gru_bih, gru_bhh,
           lstm_Wih0, lstm_Whh0, lstm_bih0, lstm_bhh0,
           lstm_Wih1, lstm_Whh1, lstm_bih1, lstm_bhh1,
           lstm_Wih2, lstm_Whh2, lstm_bih2, lstm_bhh2,
           W_pred, b_pred):
    src = edge_index[0].astype(jnp.int32)
    dst = edge_index[1].astype(jnp.int32)
    row_idx = (etypes.astype(jnp.int32) * N + src).reshape(NCHUNK, CHB)
    dst2d = dst.reshape(NCHUNK, CHB)

    w_kT = jnp.transpose(W_et, (0, 2, 1))
    wiT = gru_Wih.T
    whT = gru_Whh.T
    bi = gru_bih.reshape(1, 3 * F)
    bh = gru_bhh.reshape(1, 3 * F)
    zeros_nf = jnp.zeros((RPT, F), _f32)

    h = feats
    for _ in range(N_STEPS):
        h_all = _hall(h, w_kT, b_et.reshape(K, 1, F))
        parts = _edge_pass(h_all, row_idx, dst2d, zeros_nf)
        h = _gru(parts, h, wiT, whT, bi, bh)

    lstm_wT = [(lstm_Wih0.T, lstm_Whh0.T), (lstm_Wih1.T, lstm_Whh1.T),
               (lstm_Wih2.T, lstm_Whh2.T)]
    lstm_b = [(lstm_bih0 + lstm_bhh0).reshape(1, 4 * HID),
              (lstm_bih1 + lstm_bhh1).reshape(1, 4 * HID),
              (lstm_bih2 + lstm_bhh2).reshape(1, 4 * HID)]
    ids2d = node_graph_ids.astype(jnp.int32).reshape(N, 1)
    return _s2s(h, ids2d, lstm_wT, lstm_b, W_pred.T, b_pred.reshape(1, 3))
